# Initial kernel scaffold; baseline (speedup 1.0000x reference)
import jax
import jax.numpy as jnp
from jax.experimental import pallas as pl


def kernel(inputs, weight):
    raise NotImplementedError("write your pallas kernel here")


# speedup vs baseline: 1.0555x; 1.0555x over previous
import jax
import jax.numpy as jnp
from jax import lax
from jax.experimental import pallas as pl
from jax.experimental.pallas import tpu as pltpu
from jax.experimental.pallas import tpu_sc as plsc

_K = 8192
_D = 32
_N = 8192
_BM = 256
_G = _N // _BM
_COMMIT = 0.25

_NC = 2
_NS = 16
_NW = _NC * _NS
_BPW = _N // _NW
_L = 16


def _vq_tc_body(x_ref, wt_ref, xsq_ref, wsq_ref,
                idx_ref, loss_ref, ppl_ref, counts_ref):
    step = pl.program_id(0)
    x = x_ref[...]
    s = lax.dot_general(x, wt_ref[...], (((1,), (0,)), ((), ())),
                        preferred_element_type=jnp.float32)
    d = (xsq_ref[...] + wsq_ref[...]) - 2.0 * s
    minval = jnp.min(d, axis=1, keepdims=True)
    ii = lax.broadcasted_iota(jnp.int32, (_BM, _K), 1)
    idx = jnp.min(jnp.where(d == minval, ii, _K), axis=1)
    idx_ref[0, 0, :] = idx
    onehot = (ii == idx[:, None]).astype(jnp.float32)

    @pl.when(step == 0)
    def _init():
        counts_ref[...] = jnp.zeros_like(counts_ref)
        loss_ref[...] = jnp.zeros_like(loss_ref)

    counts_ref[...] += jnp.sum(onehot, axis=0, keepdims=True)
    loss_ref[...] += jnp.reshape(jnp.sum(minval), (1, 1))

    @pl.when(step == _G - 1)
    def _finish():
        p = counts_ref[...] * (1.0 / _N)
        ent = jnp.sum(p * jnp.log(p + 1e-10))
        ppl_ref[...] = jnp.reshape(jnp.exp(-ent), (1, 1))
        loss_ref[...] = loss_ref[...] * (_COMMIT / (_N * _D))


def _vq_distances_argmin(flat, wt, xsq, wsq):
    return pl.pallas_call(
        _vq_tc_body,
        grid=(_G,),
        in_specs=[
            pl.BlockSpec((_BM, _D), lambda i: (i, 0)),
            pl.BlockSpec((_D, _K), lambda i: (0, 0)),
            pl.BlockSpec((_BM, 1), lambda i: (i, 0)),
            pl.BlockSpec((1, _K), lambda i: (0, 0)),
        ],
        out_specs=[
            pl.BlockSpec((1, 1, _BM), lambda i: (i, 0, 0)),
            pl.BlockSpec((1, 1), lambda i: (0, 0)),
            pl.BlockSpec((1, 1), lambda i: (0, 0)),
        ],
        out_shape=[
            jax.ShapeDtypeStruct((_G, 1, _BM), jnp.int32),
            jax.ShapeDtypeStruct((1, 1), jnp.float32),
            jax.ShapeDtypeStruct((1, 1), jnp.float32),
        ],
        scratch_shapes=[pltpu.VMEM((1, _K), jnp.float32)],
    )(flat, wt, xsq, wsq)


_WP = 128
_CH = 128
_NCHUNK = _BPW // _CH


def _sc_gather_body(idx_hbm, x_hbm, wpad_hbm, out_hbm, idx_v, w_v, x_v, sem):
    wid = lax.axis_index("s") * _NC + lax.axis_index("c")
    base = wid * _BPW
    for c in range(_NCHUNK):
        pltpu.sync_copy(idx_hbm.at[pl.ds(base + c * _CH, _CH)], idx_v.at[c])
    copies = [
        pltpu.async_copy(wpad_hbm.at[idx_v.at[c]],
                         w_v.at[pl.ds(c * _CH, _CH)], sem)
        for c in range(_NCHUNK)
    ]
    pltpu.sync_copy(x_hbm.at[pl.ds(base * _D, _BPW * _D)], x_v)
    for cp in copies:
        cp.wait()

    def body(r, carry):
        for j in range(_D // _L):
            xv = x_v[pl.ds(r * _D + j * _L, _L)]
            wv = w_v[r, pl.ds(j * _L, _L)]
            x_v[pl.ds(r * _D + j * _L, _L)] = xv + (wv - xv)
        return carry

    lax.fori_loop(0, _BPW, body, 0)
    pltpu.sync_copy(x_v, out_hbm.at[pl.ds(base * _D, _BPW * _D)])


def _sc_gather_combine(idx_flat, x_flat, wpad):
    fn = pl.kernel(
        _sc_gather_body,
        out_type=jax.ShapeDtypeStruct((_N * _D,), jnp.float32),
        mesh=plsc.VectorSubcoreMesh(core_axis_name="c", subcore_axis_name="s"),
        scratch_types=[
            pltpu.VMEM((_NCHUNK, _CH), jnp.int32),
            pltpu.VMEM((_BPW, _WP), jnp.float32),
            pltpu.VMEM((_BPW * _D,), jnp.float32),
            pltpu.SemaphoreType.DMA,
        ],
    )
    return fn(idx_flat, x_flat, wpad)


def kernel(inputs, weight):
    B, D, H, W = inputs.shape
    flat = jnp.transpose(inputs, (0, 2, 3, 1)).reshape(-1, D)
    xsq = jnp.sum(flat ** 2, axis=1, keepdims=True)
    wsq = jnp.sum(weight ** 2, axis=1)[None, :]
    wt = weight.T

    idx3, loss11, ppl11 = _vq_distances_argmin(flat, wt, xsq, wsq)
    idx_flat = idx3.reshape(-1)

    wpad = jnp.pad(weight, ((0, 0), (0, _WP - _D)))
    combined = _sc_gather_combine(idx_flat, flat.reshape(-1), wpad)
    quantized = jnp.transpose(combined.reshape(B, H, W, D), (0, 3, 1, 2))
    return (quantized, loss11[0, 0], idx_flat.reshape(B, H, W), ppl11[0, 0])


# speedup vs baseline: 1.1651x; 1.1038x over previous
import jax
import jax.numpy as jnp
from jax import lax
from jax.experimental import pallas as pl
from jax.experimental.pallas import tpu as pltpu
from jax.experimental.pallas import tpu_sc as plsc

_K = 8192
_D = 32
_N = 8192
_BM = 512
_G = _N // _BM
_COMMIT = 0.25

_NC = 2
_NS = 16
_NW = _NC * _NS
_BPW = _N // _NW
_L = 16


def _vq_tc_body(x_ref, wt_ref, xsq_ref, wsq_ref, hilo_ref,
                idx_ref, loss_ref, ppl_ref, counts_ref):
    step = pl.program_id(0)
    x = x_ref[...]
    s = lax.dot_general(x, wt_ref[...], (((1,), (0,)), ((), ())),
                        preferred_element_type=jnp.float32)
    m = wsq_ref[...] - 2.0 * s
    minval = jnp.min(m, axis=1, keepdims=True)
    eq = m == minval
    idxf = jnp.min(jnp.where(eq, hilo_ref[...], 3.0e7), axis=1, keepdims=True)
    idx_ref[0, :, :] = idxf.astype(jnp.int32)
    onehot = jnp.where(eq, 1.0, 0.0)

    @pl.when(step == 0)
    def _init():
        counts_ref[...] = jnp.zeros_like(counts_ref)
        loss_ref[...] = jnp.zeros_like(loss_ref)

    counts_ref[...] += jnp.sum(onehot, axis=0, keepdims=True)
    loss_ref[...] += jnp.reshape(jnp.sum(minval) + jnp.sum(xsq_ref[...]), (1, 1))

    @pl.when(step == _G - 1)
    def _finish():
        p = counts_ref[...] * (1.0 / _N)
        ent = jnp.sum(p * jnp.log(p + 1e-10))
        ppl_ref[...] = jnp.reshape(jnp.exp(-ent), (1, 1))
        loss_ref[...] = loss_ref[...] * (_COMMIT / (_N * _D))


def _vq_distances_argmin(flat, wt, xsq, wsq, hilo):
    return pl.pallas_call(
        _vq_tc_body,
        grid=(_G,),
        in_specs=[
            pl.BlockSpec((_BM, _D), lambda i: (i, 0)),
            pl.BlockSpec((_D, _K), lambda i: (0, 0)),
            pl.BlockSpec((_BM, 1), lambda i: (i, 0)),
            pl.BlockSpec((1, _K), lambda i: (0, 0)),
            pl.BlockSpec((1, _K), lambda i: (0, 0)),
        ],
        out_specs=[
            pl.BlockSpec((1, _BM, 1), lambda i: (i, 0, 0)),
            pl.BlockSpec((1, 1), lambda i: (0, 0)),
            pl.BlockSpec((1, 1), lambda i: (0, 0)),
        ],
        out_shape=[
            jax.ShapeDtypeStruct((_G, _BM, 1), jnp.int32),
            jax.ShapeDtypeStruct((1, 1), jnp.float32),
            jax.ShapeDtypeStruct((1, 1), jnp.float32),
        ],
        scratch_shapes=[pltpu.VMEM((1, _K), jnp.float32)],
    )(flat, wt, xsq, wsq, hilo)


_WP = 128
_CH = 128
_NCHUNK = _BPW // _CH


def _sc_gather_body(idx_hbm, x_hbm, wpad_hbm, out_hbm, idx_v, w_v, x_v, sem):
    wid = lax.axis_index("s") * _NC + lax.axis_index("c")
    base = wid * _BPW
    for c in range(_NCHUNK):
        pltpu.sync_copy(idx_hbm.at[pl.ds(base + c * _CH, _CH)], idx_v.at[c])
    copies = [
        pltpu.async_copy(wpad_hbm.at[idx_v.at[c]],
                         w_v.at[pl.ds(c * _CH, _CH)], sem)
        for c in range(_NCHUNK)
    ]
    pltpu.sync_copy(x_hbm.at[pl.ds(base * _D, _BPW * _D)], x_v)
    for cp in copies:
        cp.wait()

    def body(r, carry):
        for j in range(_D // _L):
            xv = x_v[pl.ds(r * _D + j * _L, _L)]
            wv = w_v[r, pl.ds(j * _L, _L)]
            x_v[pl.ds(r * _D + j * _L, _L)] = xv + (wv - xv)
        return carry

    lax.fori_loop(0, _BPW, body, 0)
    pltpu.sync_copy(x_v, out_hbm.at[pl.ds(base * _D, _BPW * _D)])


def _sc_gather_combine(idx_flat, x_flat, wpad):
    fn = pl.kernel(
        _sc_gather_body,
        out_type=jax.ShapeDtypeStruct((_N * _D,), jnp.float32),
        mesh=plsc.VectorSubcoreMesh(core_axis_name="c", subcore_axis_name="s"),
        scratch_types=[
            pltpu.VMEM((_NCHUNK, _CH), jnp.int32),
            pltpu.VMEM((_BPW, _WP), jnp.float32),
            pltpu.VMEM((_BPW * _D,), jnp.float32),
            pltpu.SemaphoreType.DMA,
        ],
    )
    return fn(idx_flat, x_flat, wpad)


def kernel(inputs, weight):
    B, D, H, W = inputs.shape
    flat = jnp.transpose(inputs, (0, 2, 3, 1)).reshape(-1, D)
    xsq = jnp.sum(flat ** 2, axis=1, keepdims=True)
    wsq = jnp.sum(weight ** 2, axis=1)[None, :]
    wt = weight.T
    hilo = jnp.arange(_K, dtype=jnp.float32)[None, :]

    idx3, loss11, ppl11 = _vq_distances_argmin(flat, wt, xsq, wsq, hilo)
    idx_flat = idx3.reshape(-1)

    wpad = jnp.pad(weight, ((0, 0), (0, _WP - _D)))
    combined = _sc_gather_combine(idx_flat, flat.reshape(-1), wpad)
    quantized = jnp.transpose(combined.reshape(B, H, W, D), (0, 3, 1, 2))
    return (quantized, loss11[0, 0], idx_flat.reshape(B, H, W), ppl11[0, 0])


# speedup vs baseline: 1.1737x; 1.0074x over previous
import jax
import jax.numpy as jnp
from jax import lax
from jax.experimental import pallas as pl
from jax.experimental.pallas import tpu as pltpu
from jax.experimental.pallas import tpu_sc as plsc

_K = 8192
_D = 32
_N = 8192
_BM = 1024
_G = _N // _BM
_COMMIT = 0.25

_NC = 2
_NS = 16
_NW = _NC * _NS
_BPW = _N // _NW
_L = 16


def _vq_tc_body(x_ref, wt_ref, xsq_ref, wsq_ref, hilo_ref,
                idx_ref, loss_ref, ppl_ref, counts_ref):
    step = pl.program_id(0)
    x = x_ref[...]
    s = lax.dot_general(x, wt_ref[...], (((1,), (0,)), ((), ())),
                        preferred_element_type=jnp.float32)
    m = wsq_ref[...] - 2.0 * s
    minval = jnp.min(m, axis=1, keepdims=True)
    eq = m == minval
    idxf = jnp.min(jnp.where(eq, hilo_ref[...], 3.0e7), axis=1, keepdims=True)
    idx_ref[0, :, :] = idxf.astype(jnp.int32)
    onehot = jnp.where(eq, 1.0, 0.0)

    @pl.when(step == 0)
    def _init():
        counts_ref[...] = jnp.zeros_like(counts_ref)
        loss_ref[...] = jnp.zeros_like(loss_ref)

    counts_ref[...] += jnp.sum(onehot, axis=0, keepdims=True)
    loss_ref[...] += jnp.reshape(jnp.sum(minval) + jnp.sum(xsq_ref[...]), (1, 1))

    @pl.when(step == _G - 1)
    def _finish():
        p = counts_ref[...] * (1.0 / _N)
        ent = jnp.sum(p * jnp.log(p + 1e-10))
        ppl_ref[...] = jnp.reshape(jnp.exp(-ent), (1, 1))
        loss_ref[...] = loss_ref[...] * (_COMMIT / (_N * _D))


def _vq_distances_argmin(flat, wt, xsq, wsq, hilo):
    return pl.pallas_call(
        _vq_tc_body,
        grid=(_G,),
        in_specs=[
            pl.BlockSpec((_BM, _D), lambda i: (i, 0)),
            pl.BlockSpec((_D, _K), lambda i: (0, 0)),
            pl.BlockSpec((_BM, 1), lambda i: (i, 0)),
            pl.BlockSpec((1, _K), lambda i: (0, 0)),
            pl.BlockSpec((1, _K), lambda i: (0, 0)),
        ],
        out_specs=[
            pl.BlockSpec((1, _BM, 1), lambda i: (i, 0, 0)),
            pl.BlockSpec((1, 1), lambda i: (0, 0)),
            pl.BlockSpec((1, 1), lambda i: (0, 0)),
        ],
        out_shape=[
            jax.ShapeDtypeStruct((_G, _BM, 1), jnp.int32),
            jax.ShapeDtypeStruct((1, 1), jnp.float32),
            jax.ShapeDtypeStruct((1, 1), jnp.float32),
        ],
        scratch_shapes=[pltpu.VMEM((1, _K), jnp.float32)],
    )(flat, wt, xsq, wsq, hilo)


_WP = 128
_CH = 128
_NCHUNK = _BPW // _CH


def _sc_gather_body(idx_hbm, x_hbm, wpad_hbm, out_hbm, idx_v, w_v, x_v, sem):
    wid = lax.axis_index("s") * _NC + lax.axis_index("c")
    base = wid * _BPW
    for c in range(_NCHUNK):
        pltpu.sync_copy(idx_hbm.at[pl.ds(base + c * _CH, _CH)], idx_v.at[c])
    copies = [
        pltpu.async_copy(wpad_hbm.at[idx_v.at[c]],
                         w_v.at[pl.ds(c * _CH, _CH)], sem)
        for c in range(_NCHUNK)
    ]
    pltpu.sync_copy(x_hbm.at[pl.ds(base * _D, _BPW * _D)], x_v)
    for cp in copies:
        cp.wait()

    def body(r, carry):
        for j in range(_D // _L):
            xv = x_v[pl.ds(r * _D + j * _L, _L)]
            wv = w_v[r, pl.ds(j * _L, _L)]
            x_v[pl.ds(r * _D + j * _L, _L)] = xv + (wv - xv)
        return carry

    lax.fori_loop(0, _BPW, body, 0)
    pltpu.sync_copy(x_v, out_hbm.at[pl.ds(base * _D, _BPW * _D)])


def _sc_gather_combine(idx_flat, x_flat, wpad):
    fn = pl.kernel(
        _sc_gather_body,
        out_type=jax.ShapeDtypeStruct((_N * _D,), jnp.float32),
        mesh=plsc.VectorSubcoreMesh(core_axis_name="c", subcore_axis_name="s"),
        scratch_types=[
            pltpu.VMEM((_NCHUNK, _CH), jnp.int32),
            pltpu.VMEM((_BPW, _WP), jnp.float32),
            pltpu.VMEM((_BPW * _D,), jnp.float32),
            pltpu.SemaphoreType.DMA,
        ],
    )
    return fn(idx_flat, x_flat, wpad)


def kernel(inputs, weight):
    B, D, H, W = inputs.shape
    flat = jnp.transpose(inputs, (0, 2, 3, 1)).reshape(-1, D)
    xsq = jnp.sum(flat ** 2, axis=1, keepdims=True)
    wsq = jnp.sum(weight ** 2, axis=1)[None, :]
    wt = weight.T
    hilo = jnp.arange(_K, dtype=jnp.float32)[None, :]

    idx3, loss11, ppl11 = _vq_distances_argmin(flat, wt, xsq, wsq, hilo)
    idx_flat = idx3.reshape(-1)

    wpad = jnp.pad(weight, ((0, 0), (0, _WP - _D)))
    combined = _sc_gather_combine(idx_flat, flat.reshape(-1), wpad)
    quantized = jnp.transpose(combined.reshape(B, H, W, D), (0, 3, 1, 2))
    return (quantized, loss11[0, 0], idx_flat.reshape(B, H, W), ppl11[0, 0])
